# baseline (speedup 1.0000x reference)
import jax
import jax.numpy as jnp
from jax import lax
from jax.experimental import pallas as pl
from jax.experimental.pallas import tpu as pltpu
from jax.experimental.pallas import tpu_sc as plsc

B = 64
DIM = 8192
MAX_VAL = 32
NSC = 8
NTC = B - NSC
NC = 2
NS = 16
NW = NC * NS
SPLIT = NW // NSC
SEG = DIM // SPLIT
E = 512
CW = E * MAX_VAL
N_SEG_CHUNKS = SEG // E
GROUPS = E // 16
NBUF = 4


def _sc_body(x_hbm, g_hbm, u_hbm, w_hbm, out_hbm,
             gbuf0, gbuf1, gbuf2, gbuf3, rowbuf, xbuf, wbuf, ubuf, pbuf,
             rbuf, shared, sem0, sem1, sem2, sem3):
    c = lax.axis_index("c")
    s = lax.axis_index("s")
    wid2 = c * NS + s
    row = wid2 // SPLIT
    seg = wid2 % SPLIT
    d0 = seg * SEG
    pltpu.sync_copy(w_hbm.at[pl.ds(d0, SEG)], wbuf)
    pltpu.sync_copy(u_hbm, ubuf)
    iota = lax.iota(jnp.int32, 16)
    pcs = [iota * MAX_VAL + ((iota + k) & (MAX_VAL - 1)) for k in range(MAX_VAL)]

    def compute_chunk(gbuf, ci):
        def group_body(gi, _):
            gslice = gbuf.at[pl.ds(gi * 16 * MAX_VAL, 16 * MAX_VAL)]
            best = plsc.load_gather(gslice, [pcs[0]])
            bpc = pcs[0]
            for k in range(1, MAX_VAL):
                dk = plsc.load_gather(gslice, [pcs[k]])
                take = dk > best
                bpc = jnp.where(take, pcs[k], bpc)
                best = jnp.maximum(dk, best)
            cand = (bpc & (MAX_VAL - 1)).astype(jnp.float32)
            rowbuf[pl.ds(ci * E + gi * 16, 16)] = cand
            return 0

        lax.fori_loop(0, GROUPS, group_body, 0)

    gbufs = [gbuf0, gbuf1, gbuf2, gbuf3]
    sems = [sem0, sem1, sem2, sem3]
    base_w = d0 * MAX_VAL

    for j in range(NBUF):
        pltpu.async_copy(g_hbm.at[row, pl.ds(base_w + j * CW, CW)],
                         gbufs[j], sems[j])

    def super_body(sp, _):
        for j in range(NBUF):
            ci = sp * NBUF + j
            pltpu.make_async_copy(
                g_hbm.at[row, pl.ds(0, CW)], gbufs[j], sems[j]).wait()
            compute_chunk(gbufs[j], ci)

            @pl.when(ci + NBUF < N_SEG_CHUNKS)
            def _(ci=ci, j=j):
                pltpu.async_copy(
                    g_hbm.at[row, pl.ds(base_w + (ci + NBUF) * CW, CW)],
                    gbufs[j], sems[j])

        return 0

    lax.fori_loop(0, N_SEG_CHUNKS // NBUF, super_body, 0)

    pltpu.sync_copy(x_hbm.at[row, pl.ds(d0, SEG)], xbuf)

    def dot_body(j, accv):
        nv = rowbuf[pl.ds(j * 16, 16)]
        xv = xbuf[pl.ds(j * 16, 16)]
        wv = wbuf[pl.ds(j * 16, 16)]
        return accv + (nv - xv) * wv

    accv = lax.fori_loop(0, SEG // 16, dot_body, jnp.zeros((16,), jnp.float32))
    pbuf[...] = accv
    pltpu.sync_copy(pbuf, shared.at[s])
    plsc.subcore_barrier()
    g0 = (s // SPLIT) * SPLIT
    pltpu.sync_copy(shared.at[pl.ds(g0, SPLIT)], rbuf)
    tot = jnp.zeros((16,), jnp.float32)
    for t in range(SPLIT):
        tot = tot + rbuf[t, pl.ds(0, 16)]
    diff = jnp.sum(tot)
    la = jnp.exp(jnp.full((16,), diff))
    ub = plsc.load_gather(ubuf, [jnp.full((16,), row, jnp.int32)])
    accept = la > ub

    def blend_body(j, _):
        nv = rowbuf[pl.ds(j * 16, 16)]
        xv = xbuf[pl.ds(j * 16, 16)]
        rowbuf[pl.ds(j * 16, 16)] = jnp.where(accept, nv, xv)
        return 0

    lax.fori_loop(0, SEG // 16, blend_body, 0)
    pltpu.sync_copy(rowbuf, out_hbm.at[row, pl.ds(d0, SEG)])


def _sc_run(x, g2, u, w):
    mesh = plsc.VectorSubcoreMesh(core_axis_name="c", subcore_axis_name="s",
                                  num_cores=NC, num_subcores=NS)
    run = pl.kernel(
        _sc_body,
        out_type=jax.ShapeDtypeStruct((NSC, DIM), jnp.float32),
        mesh=mesh,
        compiler_params=pltpu.CompilerParams(needs_layout_passes=False),
        scratch_types=(
            [pltpu.VMEM((CW,), jnp.float32)] * NBUF
            + [
                pltpu.VMEM((SEG,), jnp.float32),
                pltpu.VMEM((SEG,), jnp.float32),
                pltpu.VMEM((SEG,), jnp.float32),
                pltpu.VMEM((B,), jnp.float32),
                pltpu.VMEM((16,), jnp.float32),
                pltpu.VMEM((SPLIT, 16), jnp.float32),
                pltpu.VMEM_SHARED((NS, 16), jnp.float32),
            ]
            + [pltpu.SemaphoreType.DMA] * NBUF
        ),
    )
    return run(x, g2, u, w)


TC_RB = 8
TC_DC = 512
TC_NC = DIM // TC_DC


def _tc_prop_body(x_ref, g_ref, w_ref, nc_ref, diff_ref, acc_ref):
    ci = pl.program_id(1)

    @pl.when(ci == 0)
    def _():
        acc_ref[...] = jnp.zeros_like(acc_ref)

    idx = jnp.argmax(g_ref[...], axis=-1)
    nc = idx.astype(jnp.float32)
    nc_ref[...] = nc
    part = (nc - x_ref[...]) * w_ref[...][None, :]
    acc_ref[...] += jnp.sum(part.reshape(TC_RB, TC_DC // 128, 128), axis=1)

    @pl.when(ci == TC_NC - 1)
    def _():
        diff_ref[...] = jnp.sum(acc_ref[...], axis=1, keepdims=True)


def _tc_blend_body(x_ref, nc_ref, u_ref, diff_ref, o_ref):
    accept = jnp.exp(diff_ref[...][:, 0]) > u_ref[...][:, 0]
    o_ref[...] = jnp.where(accept[:, None], nc_ref[...], x_ref[...])


def _tc_run(x, g, u2, w):
    nc, diff = pl.pallas_call(
        _tc_prop_body,
        grid=(NTC // TC_RB, TC_NC),
        in_specs=[
            pl.BlockSpec((TC_RB, TC_DC), lambda i, c: (NSC // TC_RB + i, c)),
            pl.BlockSpec((TC_RB, TC_DC, MAX_VAL),
                         lambda i, c: (NSC // TC_RB + i, c, 0)),
            pl.BlockSpec((TC_DC,), lambda i, c: (c,)),
        ],
        out_specs=[
            pl.BlockSpec((TC_RB, TC_DC), lambda i, c: (i, c)),
            pl.BlockSpec((TC_RB, 1), lambda i, c: (i, 0)),
        ],
        out_shape=[
            jax.ShapeDtypeStruct((NTC, DIM), jnp.float32),
            jax.ShapeDtypeStruct((NTC, 1), jnp.float32),
        ],
        scratch_shapes=[pltpu.VMEM((TC_RB, 128), jnp.float32)],
    )(x, g, w)
    return pl.pallas_call(
        _tc_blend_body,
        grid=(NTC // TC_RB,),
        in_specs=[
            pl.BlockSpec((TC_RB, DIM), lambda i: (NSC // TC_RB + i, 0)),
            pl.BlockSpec((TC_RB, DIM), lambda i: (i, 0)),
            pl.BlockSpec((TC_RB, 1), lambda i: (NSC // TC_RB + i, 0)),
            pl.BlockSpec((TC_RB, 1), lambda i: (i, 0)),
        ],
        out_specs=pl.BlockSpec((TC_RB, DIM), lambda i: (i, 0)),
        out_shape=jax.ShapeDtypeStruct((NTC, DIM), jnp.float32),
    )(x, nc, u2, diff)


@jax.jit
def kernel(x, g, u, w):
    g2 = g[:NSC].reshape(NSC, DIM * MAX_VAL)
    out_sc = _sc_run(x[:NSC], g2, u, w)
    out_tc = _tc_run(x, g, u.reshape(B, 1), w)
    return jnp.concatenate([out_sc, out_tc], axis=0)


# speedup vs baseline: 1.6799x; 1.6799x over previous
import jax
import jax.numpy as jnp
from jax import lax
from jax.experimental import pallas as pl
from jax.experimental.pallas import tpu as pltpu
from jax.experimental.pallas import tpu_sc as plsc

B = 64
DIM = 8192
MAX_VAL = 32
NC = 2
NS = 16
NW = NC * NS
ROWS_PER_W = B // NW
E = 512
CW = E * MAX_VAL
N_CHUNKS = DIM // E
GROUPS = E // 16
NBUF = 4


def _body(x_hbm, g_hbm, u_hbm, w_hbm, out_hbm,
          gbuf0, gbuf1, gbuf2, gbuf3, rowbuf, xbuf, wbuf, ubuf,
          sem0, sem1, sem2, sem3):
    wid = lax.axis_index("s") * NC + lax.axis_index("c")
    pltpu.sync_copy(w_hbm, wbuf)
    pltpu.sync_copy(u_hbm, ubuf)
    iota = lax.iota(jnp.int32, 16)
    pcs = [iota * MAX_VAL + ((iota + k) & (MAX_VAL - 1)) for k in range(MAX_VAL)]

    def compute_chunk(gbuf, ci):
        def group_body(gi, _):
            gslice = gbuf.at[pl.ds(gi * 16 * MAX_VAL, 16 * MAX_VAL)]
            best = plsc.load_gather(gslice, [pcs[0]])
            bpc = pcs[0]
            for k in range(1, MAX_VAL):
                dk = plsc.load_gather(gslice, [pcs[k]])
                take = (dk > best) | ((dk == best) & (pcs[k] < bpc))
                bpc = jnp.where(take, pcs[k], bpc)
                best = jnp.maximum(dk, best)
            cand = (bpc & (MAX_VAL - 1)).astype(jnp.float32)
            rowbuf[pl.ds(ci * E + gi * 16, 16)] = cand
            return 0

        lax.fori_loop(0, GROUPS, group_body, 0)

    gbufs = [gbuf0, gbuf1, gbuf2, gbuf3]
    sems = [sem0, sem1, sem2, sem3]

    for r in range(ROWS_PER_W):
        b = wid * ROWS_PER_W + r
        for j in range(NBUF):
            pltpu.async_copy(g_hbm.at[b, pl.ds(j * CW, CW)], gbufs[j], sems[j])

        def super_body(sp, _, b=b):
            for j in range(NBUF):
                ci = sp * NBUF + j
                pltpu.make_async_copy(
                    g_hbm.at[b, pl.ds(0, CW)], gbufs[j], sems[j]).wait()
                compute_chunk(gbufs[j], ci)

                @pl.when(ci + NBUF < N_CHUNKS)
                def _(ci=ci, j=j):
                    pltpu.async_copy(
                        g_hbm.at[b, pl.ds((ci + NBUF) * CW, CW)],
                        gbufs[j], sems[j])

            return 0

        lax.fori_loop(0, N_CHUNKS // NBUF, super_body, 0)

        pltpu.sync_copy(x_hbm.at[b], xbuf)

        def dot_body(j, accv):
            nv = rowbuf[pl.ds(j * 16, 16)]
            xv = xbuf[pl.ds(j * 16, 16)]
            wv = wbuf[pl.ds(j * 16, 16)]
            return accv + (nv - xv) * wv

        accv = lax.fori_loop(0, DIM // 16, dot_body,
                             jnp.zeros((16,), jnp.float32))
        diff = jnp.sum(accv)
        la = jnp.exp(jnp.full((16,), diff))
        ub = plsc.load_gather(ubuf, [jnp.full((16,), b, jnp.int32)])
        accept = la > ub

        def blend_body(j, _):
            nv = rowbuf[pl.ds(j * 16, 16)]
            xv = xbuf[pl.ds(j * 16, 16)]
            rowbuf[pl.ds(j * 16, 16)] = jnp.where(accept, nv, xv)
            return 0

        lax.fori_loop(0, DIM // 16, blend_body, 0)
        pltpu.sync_copy(rowbuf, out_hbm.at[b])


@jax.jit
def kernel(x, g, u, w):
    g2 = g.reshape(B, DIM * MAX_VAL)
    mesh = plsc.VectorSubcoreMesh(core_axis_name="c", subcore_axis_name="s",
                                  num_cores=NC, num_subcores=NS)
    run = pl.kernel(
        _body,
        out_type=jax.ShapeDtypeStruct((B, DIM), jnp.float32),
        mesh=mesh,
        compiler_params=pltpu.CompilerParams(needs_layout_passes=False),
        scratch_types=(
            [pltpu.VMEM((CW,), jnp.float32)] * NBUF
            + [
                pltpu.VMEM((DIM,), jnp.float32),
                pltpu.VMEM((DIM,), jnp.float32),
                pltpu.VMEM((DIM,), jnp.float32),
                pltpu.VMEM((B,), jnp.float32),
            ]
            + [pltpu.SemaphoreType.DMA] * NBUF
        ),
    )
    return run(x, g2, u, w)
